# baseline (speedup 1.0000x reference)
import jax
import jax.numpy as jnp
from jax import lax
from jax.experimental import pallas as pl
from jax.experimental.pallas import tpu as pltpu
from jax.experimental.pallas import tpu_sc as plsc

B, H, S, D = 8, 4, 2048, 128
SW = 512
BH = B * H
CACHE_ROWS = BH * S


def _sc_body(k0, v0, k1, v1, lat, pos_hbm, out, pos_v, sem, sem2):
    w = lax.axis_index("s") * 2 + lax.axis_index("c")
    row_lo = w * S

    cps = [
        pltpu.make_async_copy(
            src.at[pl.ds(row_lo, S)],
            out.at[pl.ds(c * CACHE_ROWS + row_lo, S)],
            sem,
        )
        for c, src in enumerate((k0, v0, k1, v1))
    ]
    for cp in cps:
        cp.start()

    pltpu.sync_copy(pos_hbm, pos_v)
    pv = pos_v[pl.ds(w // H, 16)]
    p0 = pv[0]
    p1 = lax.bitwise_and(p0, SW - 1)

    for cp in cps:
        cp.wait()

    rcps = [
        pltpu.make_async_copy(
            lat.at[pl.ds(c * BH + w, 1)],
            out.at[pl.ds(c * CACHE_ROWS + row_lo + (p0 if c < 2 else p1), 1)],
            sem2,
        )
        for c in range(4)
    ]
    for cp in rcps:
        cp.start()
    for cp in rcps:
        cp.wait()


def kernel(k_cache_0, v_cache_0, k_cache_1, v_cache_1,
           latest_k_0, latest_v_0, latest_k_1, latest_v_1, position_ids):
    caches = [cc.reshape(BH * S, D)
              for cc in (k_cache_0, v_cache_0, k_cache_1, v_cache_1)]
    lat = jnp.stack([latest_k_0, latest_v_0, latest_k_1, latest_v_1],
                    axis=0).reshape(4 * BH, D)
    pos = jnp.pad(position_ids.reshape(B).astype(jnp.int32), (0, 16))

    mesh = plsc.VectorSubcoreMesh(core_axis_name="c", subcore_axis_name="s")
    run = pl.kernel(
        _sc_body,
        out_type=jax.ShapeDtypeStruct((4 * CACHE_ROWS, D), jnp.float32),
        mesh=mesh,
        scratch_types=[
            pltpu.VMEM((24,), jnp.int32),
            pltpu.SemaphoreType.DMA,
            pltpu.SemaphoreType.DMA,
        ],
    )
    out = run(*caches, lat, pos)
    return out.reshape(4, B, H, S, D)


# speedup vs baseline: 35.4098x; 35.4098x over previous
import jax
import jax.numpy as jnp
from jax import lax
from jax.experimental import pallas as pl
from jax.experimental.pallas import tpu as pltpu
from jax.experimental.pallas import tpu_sc as plsc

B, H, S, D = 8, 4, 2048, 128
SW = 512
BH = B * H
CACHE_ROWS = BH * S
CR = 256
NCHUNK = S // CR


def _sc_body(k0, v0, k1, v1, lat, pos_hbm, out, pos_v, buf0, buf1,
             si0, si1, so0, so1, sem2):
    w = lax.axis_index("s") * 2 + lax.axis_index("c")
    row_lo = w * S
    bufs = (buf0, buf1)
    sins = (si0, si1)
    souts = (so0, so1)

    out_cps = []
    for c, src in enumerate((k0, v0, k1, v1)):
        for j in range(NCHUNK):
            i = c * NCHUNK + j
            b = i % 2
            if i >= 2:
                out_cps[i - 2].wait()
            lo = row_lo + j * CR
            in_cp = pltpu.make_async_copy(
                src.at[pl.ds(lo, CR)], bufs[b], sins[b])
            in_cp.start()
            in_cp.wait()
            o_cp = pltpu.make_async_copy(
                bufs[b], out.at[pl.ds(c * CACHE_ROWS + lo, CR)], souts[b])
            o_cp.start()
            out_cps.append(o_cp)
    out_cps[-2].wait()
    out_cps[-1].wait()

    pltpu.sync_copy(pos_hbm, pos_v)
    pv = pos_v[pl.ds(w // H, 16)]
    p0 = pv[0]
    p1 = lax.bitwise_and(p0, SW - 1)

    rcps = [
        pltpu.make_async_copy(
            lat.at[pl.ds(c * BH + w, 1)],
            out.at[pl.ds(c * CACHE_ROWS + row_lo + (p0 if c < 2 else p1), 1)],
            sem2,
        )
        for c in range(4)
    ]
    for cp in rcps:
        cp.start()
    for cp in rcps:
        cp.wait()


def kernel(k_cache_0, v_cache_0, k_cache_1, v_cache_1,
           latest_k_0, latest_v_0, latest_k_1, latest_v_1, position_ids):
    caches = [cc.reshape(BH * S, D)
              for cc in (k_cache_0, v_cache_0, k_cache_1, v_cache_1)]
    lat = jnp.stack([latest_k_0, latest_v_0, latest_k_1, latest_v_1],
                    axis=0).reshape(4 * BH, D)
    pos = jnp.pad(position_ids.reshape(B).astype(jnp.int32), (0, 16))

    mesh = plsc.VectorSubcoreMesh(core_axis_name="c", subcore_axis_name="s")
    run = pl.kernel(
        _sc_body,
        out_type=jax.ShapeDtypeStruct((4 * CACHE_ROWS, D), jnp.float32),
        mesh=mesh,
        scratch_types=[
            pltpu.VMEM((24,), jnp.int32),
            pltpu.VMEM((CR, D), jnp.float32),
            pltpu.VMEM((CR, D), jnp.float32),
            pltpu.SemaphoreType.DMA,
            pltpu.SemaphoreType.DMA,
            pltpu.SemaphoreType.DMA,
            pltpu.SemaphoreType.DMA,
            pltpu.SemaphoreType.DMA,
        ],
    )
    out = run(*caches, lat, pos)
    return out.reshape(4, B, H, S, D)
